# baseline (speedup 1.0000x reference)
import jax
import jax.numpy as jnp
from jax import lax
from jax.experimental import pallas as pl
from jax.experimental.pallas import tpu as pltpu
from jax.experimental.pallas import tpu_sc as plsc

NUM_SIGNALS = 1000000
CODE_DIM = 32
BATCH = 16384

_PACK = 128 // CODE_DIM
_NLINES = NUM_SIGNALS // _PACK

_NC = 2
_NS = 16
_NW = _NC * _NS
_BPW = BATCH // _NW
_CHUNK = 128
_NCHUNK = _BPW // _CHUNK

_NTILES = NUM_SIGNALS // 128
_TAIL = NUM_SIGNALS - _NTILES * 128
_TPW = _NTILES // _NW
_TC = 4
_NCH = _TPW // _TC
_EXTRA = _NTILES - _TPW * _NW
_CW = _TC * 128
_CWP = _CW + 1
_CR = _TC * 32


def _reformat_body(tableT_hbm, tail_hbm, scratch_hbm, blk_v, s_v,
                   in_sem, out_sem):
    wid = lax.axis_index("s") * _NC + lax.axis_index("c")
    j0 = wid * _TPW
    iota = lax.iota(jnp.int32, 16)
    rids = (iota, iota + 16)

    def start_in(u, b):
        pltpu.async_copy(
            tableT_hbm.at[:, pl.ds((j0 + u * _TC) * 128, _CW)],
            blk_v.at[b, :, pl.ds(0, _CW)], in_sem)

    def wait_in(b):
        pltpu.make_async_copy(
            tableT_hbm.at[:, pl.ds(0, _CW)],
            blk_v.at[b, :, pl.ds(0, _CW)], in_sem).wait()

    def start_out(u, b):
        pltpu.async_copy(
            s_v.at[b], scratch_hbm.at[pl.ds((j0 + u * _TC) * 32, _CR), :],
            out_sem)

    def wait_out(b):
        pltpu.make_async_copy(
            s_v.at[b], scratch_hbm.at[pl.ds(0, _CR), :], out_sem).wait()

    def compute(b):
        @plsc.parallel_loop(0, _CR, unroll=8)
        def rows(s):
            col = (s // 32) * 128 + 4 * (s % 32)
            for u in range(4):
                cid = jnp.full((16,), 0, jnp.int32) + (col + u)
                for h in range(2):
                    s_v[b, s, pl.ds(16 * (2 * u + h), 16)] = (
                        plsc.load_gather(blk_v.at[b], [rids[h], cid]))

    start_in(0, 0)
    start_in(1, 1)
    wait_in(0)
    compute(0)
    start_out(0, 0)
    start_in(2, 0)
    wait_in(1)
    compute(1)
    start_out(1, 1)
    start_in(3, 1)

    def step(g, _):
        for b in range(2):
            u = g * 2 + b
            wait_out(b)
            wait_in(b)
            compute(b)
            start_out(u, b)

            @pl.when(u + 2 <= _NCH - 1)
            def _more():
                start_in(u + 2, b)
        return _

    lax.fori_loop(1, (_NCH - 1) // 2, step, None)
    wait_out(0)
    wait_in(0)
    compute(0)
    start_out(_NCH - 1, 0)
    wait_out(1)
    wait_out(0)

    @pl.when(wid < _EXTRA)
    def _leftover():
        j = _NTILES - _EXTRA + wid
        pltpu.sync_copy(tableT_hbm.at[:, pl.ds(j * 128, 128)],
                        blk_v.at[0, :, pl.ds(0, 128)])

        def rows(s, _):
            col = 4 * s
            for u in range(4):
                cid = jnp.full((16,), 0, jnp.int32) + (col + u)
                for h in range(2):
                    s_v[0, s, pl.ds(16 * (2 * u + h), 16)] = (
                        plsc.load_gather(blk_v.at[0], [rids[h], cid]))
            return _

        lax.fori_loop(0, 32, rows, None)
        pltpu.sync_copy(s_v.at[0, pl.ds(0, 32), :],
                        scratch_hbm.at[pl.ds(j * 32, 32), :])

    @pl.when(wid == _NW - 1)
    def _tail():
        pltpu.sync_copy(tail_hbm, s_v.at[0, pl.ds(0, _TAIL // 4), :])
        pltpu.sync_copy(s_v.at[0, pl.ds(0, _TAIL // 4), :],
                        scratch_hbm.at[pl.ds(_NTILES * 32, _TAIL // 4), :])


def _gather_body(idx_hbm, table_hbm, outT_hbm,
                 idx_v, line_v, off_v, rows_v, out_v, sem):
    wid = lax.axis_index("s") * _NC + lax.axis_index("c")
    base = wid * _BPW
    pltpu.sync_copy(idx_hbm.at[pl.ds(base, _BPW)], idx_v)
    for k in range(_BPW // 16):
        v = idx_v[pl.ds(k * 16, 16)]
        line_v[k // 8, pl.ds((k * 16) % _CHUNK, 16)] = v >> 2
        off_v[pl.ds(k * 16, 16)] = (v & 3) << 5
    copies = [
        pltpu.async_copy(
            table_hbm.at[line_v.at[j]],
            rows_v.at[pl.ds(j * _CHUNK, _CHUNK)],
            sem,
        )
        for j in range(_NCHUNK)
    ]
    for c in copies:
        c.wait()

    def extract(mb, _):
        s = pl.multiple_of(mb * 16, 16)
        bvec = lax.iota(jnp.int32, 16) + mb * 16
        offv = off_v[pl.ds(s, 16)]
        for c in range(CODE_DIM):
            vals = plsc.load_gather(rows_v, [bvec, offv + c])
            out_v[c, pl.ds(s, 16)] = vals
        return _

    lax.fori_loop(0, _BPW // 16, extract, None)
    pltpu.sync_copy(out_v, outT_hbm.at[:, pl.ds(base, _BPW)])


_mesh = plsc.VectorSubcoreMesh(core_axis_name="c", subcore_axis_name="s")
_params = pltpu.CompilerParams(
    needs_layout_passes=False, use_tc_tiling_on_sc=True,
    disable_bounds_checks=True)


@jax.jit
def _run(idx, tableT, tail16):
    scratch = pl.kernel(
        _reformat_body,
        mesh=_mesh,
        out_type=jax.ShapeDtypeStruct((_NLINES, 128), jnp.float32),
        scratch_types=[
            pltpu.VMEM((2, CODE_DIM, _CWP), jnp.float32),
            pltpu.VMEM((2, _CR, 128), jnp.float32),
            pltpu.SemaphoreType.DMA,
            pltpu.SemaphoreType.DMA,
        ],
        compiler_params=_params,
    )(tableT, tail16)
    return pl.kernel(
        _gather_body,
        mesh=_mesh,
        out_type=jax.ShapeDtypeStruct((CODE_DIM, BATCH), jnp.float32),
        scratch_types=[
            pltpu.VMEM((_BPW,), jnp.int32),
            pltpu.VMEM((_NCHUNK, _CHUNK), jnp.int32),
            pltpu.VMEM((_BPW,), jnp.int32),
            pltpu.VMEM((_BPW, 128), jnp.float32),
            pltpu.VMEM((CODE_DIM, _BPW), jnp.float32),
            pltpu.SemaphoreType.DMA,
        ],
        compiler_params=_params,
    )(idx, scratch)


def kernel(signal_indices, codes):
    idx = signal_indices.astype(jnp.int32)
    tail16 = codes[_NTILES * 128:, :].reshape(_TAIL // 4, 128)
    out_t = _run(idx, codes.T, tail16)
    return out_t.T


# speedup vs baseline: 1.0080x; 1.0080x over previous
import jax
import jax.numpy as jnp
from jax import lax
from jax.experimental import pallas as pl
from jax.experimental.pallas import tpu as pltpu
from jax.experimental.pallas import tpu_sc as plsc

NUM_SIGNALS = 1000000
CODE_DIM = 32
BATCH = 16384

_PACK = 128 // CODE_DIM
_NLINES = NUM_SIGNALS // _PACK

_NC = 2
_NS = 16
_NW = _NC * _NS
_BPW = BATCH // _NW
_CHUNK = 128
_NCHUNK = _BPW // _CHUNK

_NTILES = NUM_SIGNALS // 128
_TAIL = NUM_SIGNALS - _NTILES * 128
_TPW = _NTILES // _NW
_TC = 4
_NCH = _TPW // _TC
_EXTRA = _NTILES - _TPW * _NW
_CW = _TC * 128
_CWP = _CW + 1
_CR = _TC * 32


def _reformat_body(tableT_hbm, tail_hbm, scratch_hbm, blk_v, s_v,
                   in_sem, out_sem):
    wid = lax.axis_index("s") * _NC + lax.axis_index("c")
    j0 = wid * _TPW
    iota = lax.iota(jnp.int32, 16)
    rids = (iota, iota + 16)

    def start_in(u, b):
        pltpu.async_copy(
            tableT_hbm.at[:, pl.ds((j0 + u * _TC) * 128, _CW)],
            blk_v.at[b, :, pl.ds(0, _CW)], in_sem)

    def wait_in(b):
        pltpu.make_async_copy(
            tableT_hbm.at[:, pl.ds(0, _CW)],
            blk_v.at[b, :, pl.ds(0, _CW)], in_sem).wait()

    def start_out(u, b):
        pltpu.async_copy(
            s_v.at[b], scratch_hbm.at[pl.ds((j0 + u * _TC) * 32, _CR), :],
            out_sem)

    def wait_out(b):
        pltpu.make_async_copy(
            s_v.at[b], scratch_hbm.at[pl.ds(0, _CR), :], out_sem).wait()

    def compute(b):
        @plsc.parallel_loop(0, _CR, unroll=8)
        def rows(s):
            col = (s // 32) * 128 + 4 * (s % 32)
            for u in range(4):
                cid = jnp.full((16,), 0, jnp.int32) + (col + u)
                for h in range(2):
                    s_v[b, s, pl.ds(16 * (2 * u + h), 16)] = (
                        plsc.load_gather(blk_v.at[b], [rids[h], cid]))

    start_in(0, 0)
    start_in(1, 1)
    wait_in(0)
    compute(0)
    start_out(0, 0)
    start_in(2, 0)
    wait_in(1)
    compute(1)
    start_out(1, 1)
    start_in(3, 1)

    def step(g, _):
        for b in range(2):
            u = g * 2 + b
            wait_out(b)
            wait_in(b)
            compute(b)
            start_out(u, b)

            @pl.when(u + 2 <= _NCH - 1)
            def _more():
                start_in(u + 2, b)
        return _

    lax.fori_loop(1, (_NCH - 1) // 2, step, None)
    wait_out(0)
    wait_in(0)
    compute(0)
    start_out(_NCH - 1, 0)
    wait_out(1)
    wait_out(0)

    @pl.when(wid < _EXTRA)
    def _leftover():
        j = _NTILES - _EXTRA + wid
        pltpu.sync_copy(tableT_hbm.at[:, pl.ds(j * 128, 128)],
                        blk_v.at[0, :, pl.ds(0, 128)])

        def rows(s, _):
            col = 4 * s
            for u in range(4):
                cid = jnp.full((16,), 0, jnp.int32) + (col + u)
                for h in range(2):
                    s_v[0, s, pl.ds(16 * (2 * u + h), 16)] = (
                        plsc.load_gather(blk_v.at[0], [rids[h], cid]))
            return _

        lax.fori_loop(0, 32, rows, None)
        pltpu.sync_copy(s_v.at[0, pl.ds(0, 32), :],
                        scratch_hbm.at[pl.ds(j * 32, 32), :])

    @pl.when(wid == _NW - 1)
    def _tail():
        pltpu.sync_copy(tail_hbm, s_v.at[0, pl.ds(0, _TAIL // 4), :])
        pltpu.sync_copy(s_v.at[0, pl.ds(0, _TAIL // 4), :],
                        scratch_hbm.at[pl.ds(_NTILES * 32, _TAIL // 4), :])


def _gather_body(idx_hbm, table_hbm, outT_hbm,
                 idx_v, line_v, off_v, rows_v, out_v, sem):
    wid = lax.axis_index("s") * _NC + lax.axis_index("c")
    base = wid * _BPW
    pltpu.sync_copy(idx_hbm.at[pl.ds(base, _BPW)], idx_v)
    for k in range(_BPW // 16):
        v = idx_v[pl.ds(k * 16, 16)]
        line_v[k // 8, pl.ds((k * 16) % _CHUNK, 16)] = v >> 2
        off_v[pl.ds(k * 16, 16)] = (v & 3) << 5
    copies = [
        pltpu.async_copy(
            table_hbm.at[line_v.at[j]],
            rows_v.at[pl.ds(j * _CHUNK, _CHUNK)],
            sem,
        )
        for j in range(_NCHUNK)
    ]
    for c in copies:
        c.wait()

    @plsc.parallel_loop(0, _BPW // 16, unroll=2)
    def extract(mb):
        s = pl.multiple_of(mb * 16, 16)
        bvec = lax.iota(jnp.int32, 16) + mb * 16
        offv = off_v[pl.ds(s, 16)]
        for c in range(CODE_DIM):
            vals = plsc.load_gather(rows_v, [bvec, offv + c])
            out_v[c, pl.ds(s, 16)] = vals
    pltpu.sync_copy(out_v, outT_hbm.at[:, pl.ds(base, _BPW)])


_mesh = plsc.VectorSubcoreMesh(core_axis_name="c", subcore_axis_name="s")
_params = pltpu.CompilerParams(
    needs_layout_passes=False, use_tc_tiling_on_sc=True,
    disable_bounds_checks=True)


@jax.jit
def _run(idx, tableT, tail16):
    scratch = pl.kernel(
        _reformat_body,
        mesh=_mesh,
        out_type=jax.ShapeDtypeStruct((_NLINES, 128), jnp.float32),
        scratch_types=[
            pltpu.VMEM((2, CODE_DIM, _CWP), jnp.float32),
            pltpu.VMEM((2, _CR, 128), jnp.float32),
            pltpu.SemaphoreType.DMA,
            pltpu.SemaphoreType.DMA,
        ],
        compiler_params=_params,
    )(tableT, tail16)
    return pl.kernel(
        _gather_body,
        mesh=_mesh,
        out_type=jax.ShapeDtypeStruct((CODE_DIM, BATCH), jnp.float32),
        scratch_types=[
            pltpu.VMEM((_BPW,), jnp.int32),
            pltpu.VMEM((_NCHUNK, _CHUNK), jnp.int32),
            pltpu.VMEM((_BPW,), jnp.int32),
            pltpu.VMEM((_BPW, 128), jnp.float32),
            pltpu.VMEM((CODE_DIM, _BPW), jnp.float32),
            pltpu.SemaphoreType.DMA,
        ],
        compiler_params=_params,
    )(idx, scratch)


def kernel(signal_indices, codes):
    idx = signal_indices.astype(jnp.int32)
    tail16 = codes[_NTILES * 128:, :].reshape(_TAIL // 4, 128)
    out_t = _run(idx, codes.T, tail16)
    return out_t.T


# speedup vs baseline: 1.0142x; 1.0061x over previous
import jax
import jax.numpy as jnp
from jax import lax
from jax.experimental import pallas as pl
from jax.experimental.pallas import tpu as pltpu
from jax.experimental.pallas import tpu_sc as plsc

NUM_SIGNALS = 1000000
CODE_DIM = 32
BATCH = 16384

_PACK = 128 // CODE_DIM
_NLINES = NUM_SIGNALS // _PACK

_NC = 2
_NS = 16
_NW = _NC * _NS
_BPW = BATCH // _NW
_CHUNK = 128
_NCHUNK = _BPW // _CHUNK

_NTILES = NUM_SIGNALS // 128
_TAIL = NUM_SIGNALS - _NTILES * 128
_TPW = _NTILES // _NW
_TC = 4
_NCH = _TPW // _TC
_EXTRA = _NTILES - _TPW * _NW
_CW = _TC * 128
_CWP = _CW
_CR = _TC * 32


def _reformat_body(tableT_hbm, tail_hbm, scratch_hbm, blk_v, s_v,
                   in_sem, out_sem):
    wid = lax.axis_index("s") * _NC + lax.axis_index("c")
    j0 = wid * _TPW
    iota = lax.iota(jnp.int32, 16)
    rids = (iota, iota + 16)

    def start_in(u, b):
        pltpu.async_copy(
            tableT_hbm.at[:, pl.ds((j0 + u * _TC) * 128, _CW)],
            blk_v.at[b, :, pl.ds(0, _CW)], in_sem)

    def wait_in(b):
        pltpu.make_async_copy(
            tableT_hbm.at[:, pl.ds(0, _CW)],
            blk_v.at[b, :, pl.ds(0, _CW)], in_sem).wait()

    def start_out(u, b):
        pltpu.async_copy(
            s_v.at[b], scratch_hbm.at[pl.ds((j0 + u * _TC) * 32, _CR), :],
            out_sem)

    def wait_out(b):
        pltpu.make_async_copy(
            s_v.at[b], scratch_hbm.at[pl.ds(0, _CR), :], out_sem).wait()

    def compute(b):
        @plsc.parallel_loop(0, _CR, unroll=2)
        def rows(s):
            col = (s // 32) * 128 + 4 * (s % 32)
            for u in range(4):
                cid = jnp.full((16,), 0, jnp.int32) + (col + u)
                for h in range(2):
                    s_v[b, s, pl.ds(16 * (2 * u + h), 16)] = (
                        plsc.load_gather(blk_v.at[b], [rids[h], cid]))

    start_in(0, 0)
    start_in(1, 1)
    wait_in(0)
    compute(0)
    start_out(0, 0)
    start_in(2, 0)
    wait_in(1)
    compute(1)
    start_out(1, 1)
    start_in(3, 1)

    def step(g, _):
        for b in range(2):
            u = g * 2 + b
            wait_out(b)
            wait_in(b)
            compute(b)
            start_out(u, b)

            @pl.when(u + 2 <= _NCH - 1)
            def _more():
                start_in(u + 2, b)
        return _

    lax.fori_loop(1, (_NCH - 1) // 2, step, None)
    wait_out(0)
    wait_in(0)
    compute(0)
    start_out(_NCH - 1, 0)
    wait_out(1)
    wait_out(0)

    @pl.when(wid < _EXTRA)
    def _leftover():
        j = _NTILES - _EXTRA + wid
        pltpu.sync_copy(tableT_hbm.at[:, pl.ds(j * 128, 128)],
                        blk_v.at[0, :, pl.ds(0, 128)])

        def rows(s, _):
            col = 4 * s
            for u in range(4):
                cid = jnp.full((16,), 0, jnp.int32) + (col + u)
                for h in range(2):
                    s_v[0, s, pl.ds(16 * (2 * u + h), 16)] = (
                        plsc.load_gather(blk_v.at[0], [rids[h], cid]))
            return _

        lax.fori_loop(0, 32, rows, None)
        pltpu.sync_copy(s_v.at[0, pl.ds(0, 32), :],
                        scratch_hbm.at[pl.ds(j * 32, 32), :])

    @pl.when(wid == _NW - 1)
    def _tail():
        pltpu.sync_copy(tail_hbm, s_v.at[0, pl.ds(0, _TAIL // 4), :])
        pltpu.sync_copy(s_v.at[0, pl.ds(0, _TAIL // 4), :],
                        scratch_hbm.at[pl.ds(_NTILES * 32, _TAIL // 4), :])


def _gather_body(idx_hbm, table_hbm, outT_hbm,
                 idx_v, line_v, off_v, rows_v, out_v, sem):
    wid = lax.axis_index("s") * _NC + lax.axis_index("c")
    base = wid * _BPW
    pltpu.sync_copy(idx_hbm.at[pl.ds(base, _BPW)], idx_v)
    for k in range(_BPW // 16):
        v = idx_v[pl.ds(k * 16, 16)]
        line_v[k // 8, pl.ds((k * 16) % _CHUNK, 16)] = v >> 2
        off_v[pl.ds(k * 16, 16)] = (v & 3) << 5
    copies = [
        pltpu.async_copy(
            table_hbm.at[line_v.at[j]],
            rows_v.at[pl.ds(j * _CHUNK, _CHUNK)],
            sem,
        )
        for j in range(_NCHUNK)
    ]
    for c in copies:
        c.wait()

    @plsc.parallel_loop(0, _BPW // 16, unroll=2)
    def extract(mb):
        s = pl.multiple_of(mb * 16, 16)
        bvec = lax.iota(jnp.int32, 16) + mb * 16
        offv = off_v[pl.ds(s, 16)]
        for c in range(CODE_DIM):
            vals = plsc.load_gather(rows_v, [bvec, offv + c])
            out_v[c, pl.ds(s, 16)] = vals
    pltpu.sync_copy(out_v, outT_hbm.at[:, pl.ds(base, _BPW)])


_mesh = plsc.VectorSubcoreMesh(core_axis_name="c", subcore_axis_name="s")
_params = pltpu.CompilerParams(
    needs_layout_passes=False, use_tc_tiling_on_sc=True,
    disable_bounds_checks=True)


@jax.jit
def _run(idx, tableT, tail16):
    scratch = pl.kernel(
        _reformat_body,
        mesh=_mesh,
        out_type=jax.ShapeDtypeStruct((_NLINES, 128), jnp.float32),
        scratch_types=[
            pltpu.VMEM((2, CODE_DIM, _CWP), jnp.float32),
            pltpu.VMEM((2, _CR, 128), jnp.float32),
            pltpu.SemaphoreType.DMA,
            pltpu.SemaphoreType.DMA,
        ],
        compiler_params=_params,
    )(tableT, tail16)
    return pl.kernel(
        _gather_body,
        mesh=_mesh,
        out_type=jax.ShapeDtypeStruct((CODE_DIM, BATCH), jnp.float32),
        scratch_types=[
            pltpu.VMEM((_BPW,), jnp.int32),
            pltpu.VMEM((_NCHUNK, _CHUNK), jnp.int32),
            pltpu.VMEM((_BPW,), jnp.int32),
            pltpu.VMEM((_BPW, 128), jnp.float32),
            pltpu.VMEM((CODE_DIM, _BPW), jnp.float32),
            pltpu.SemaphoreType.DMA,
        ],
        compiler_params=_params,
    )(idx, scratch)


def kernel(signal_indices, codes):
    idx = signal_indices.astype(jnp.int32)
    tail16 = codes[_NTILES * 128:, :].reshape(_TAIL // 4, 128)
    out_t = _run(idx, codes.T, tail16)
    return out_t.T
